# baseline (speedup 1.0000x reference)
import functools

import jax
import jax.numpy as jnp
from jax.experimental import pallas as pl
from jax.experimental.pallas import tpu as pltpu

N = 10000
E = 4096
D = 128
T = 5
BN = 400
NB = N // BN
BLOCKS_PER_TYPE = (N // T) // BN

_CONTRACT_ROWS = (((0,), (0,)), ((), ()))


def _sweep1_body(emb_ref, h_ref, s_ref, pw_ref, pb_ref,
                 x0_ref, hb_ref, p1_ref):
    i = pl.program_id(0)
    x0 = jax.lax.dot_general(
        emb_ref[...], pw_ref[0],
        (((1,), (0,)), ((), ())),
        preferred_element_type=jnp.float32,
        precision=jax.lax.Precision.HIGHEST) + pb_ref[0]
    x0_ref[...] = x0
    xs = (s_ref[...] * x0).astype(jnp.bfloat16)
    hb = h_ref[...].astype(jnp.bfloat16)
    hb_ref[...] = hb

    @pl.when(i == 0)
    def _():
        p1_ref[...] = jnp.zeros_like(p1_ref)

    p1_ref[...] += jax.lax.dot_general(
        hb, xs, _CONTRACT_ROWS, preferred_element_type=jnp.float32)


def _sweep2_body(hb_ref, p1_ref, de_ref, s_ref, w1_ref, b1_ref,
                 x1_ref, p2_ref, m1_scr):
    i = pl.program_id(0)

    @pl.when(i == 0)
    def _():
        m1_scr[...] = (de_ref[...] * p1_ref[...]).astype(jnp.bfloat16)
        p2_ref[...] = jnp.zeros_like(p2_ref)

    hb = hb_ref[...]
    y = jax.lax.dot_general(
        hb, m1_scr[...], (((1,), (0,)), ((), ())),
        preferred_element_type=jnp.float32)
    y = s_ref[...] * y
    x1 = jax.lax.dot_general(
        y, w1_ref[...], (((1,), (0,)), ((), ())),
        preferred_element_type=jnp.float32,
        precision=jax.lax.Precision.HIGHEST) + b1_ref[...]
    x1_ref[...] = x1
    xs1 = (s_ref[...] * x1).astype(jnp.bfloat16)
    p2_ref[...] += jax.lax.dot_general(
        hb, xs1, _CONTRACT_ROWS, preferred_element_type=jnp.float32)


def _sweep3_body(hb_ref, p2_ref, de_ref, s_ref, w2_ref, b2_ref,
                 x0_ref, x1_ref, w_ref, out_ref, m2_scr):
    i = pl.program_id(0)

    @pl.when(i == 0)
    def _():
        m2_scr[...] = (de_ref[...] * p2_ref[...]).astype(jnp.bfloat16)

    y = jax.lax.dot_general(
        hb_ref[...], m2_scr[...], (((1,), (0,)), ((), ())),
        preferred_element_type=jnp.float32)
    y = s_ref[...] * y
    x2 = jax.lax.dot_general(
        y, w2_ref[...], (((1,), (0,)), ((), ())),
        preferred_element_type=jnp.float32,
        precision=jax.lax.Precision.HIGHEST) + b2_ref[...]
    w0 = w_ref[0]
    w1 = w_ref[1]
    w2 = w_ref[2]
    out_ref[...] = w0 * x0_ref[...] + w1 * x1_ref[...] + w2 * x2


def kernel(H, Dv_inv_sqrt, De_inv, emb, projW, projB, W1, b1, W2, b2, fusion):
    s2d = Dv_inv_sqrt.reshape(N, 1)
    de2d = De_inv.reshape(E, 1)
    pb3d = projB.reshape(T, 1, D)
    b1r = b1.reshape(1, D)
    b2r = b2.reshape(1, D)
    w = jax.nn.softmax(fusion, axis=0)

    row_blk = lambda i: (i, 0)
    full2d = lambda i: (0, 0)

    x0, hb, p1 = pl.pallas_call(
        _sweep1_body,
        grid=(NB,),
        in_specs=[
            pl.BlockSpec((BN, D), row_blk),
            pl.BlockSpec((BN, E), row_blk),
            pl.BlockSpec((BN, 1), row_blk),
            pl.BlockSpec((1, D, D), lambda i: (i // BLOCKS_PER_TYPE, 0, 0)),
            pl.BlockSpec((1, 1, D), lambda i: (i // BLOCKS_PER_TYPE, 0, 0)),
        ],
        out_specs=[
            pl.BlockSpec((BN, D), row_blk),
            pl.BlockSpec((BN, E), row_blk),
            pl.BlockSpec((E, D), full2d),
        ],
        out_shape=[
            jax.ShapeDtypeStruct((N, D), jnp.float32),
            jax.ShapeDtypeStruct((N, E), jnp.bfloat16),
            jax.ShapeDtypeStruct((E, D), jnp.float32),
        ],
    )(emb, H, s2d, projW, pb3d)

    x1, p2 = pl.pallas_call(
        _sweep2_body,
        grid=(NB,),
        in_specs=[
            pl.BlockSpec((BN, E), row_blk),
            pl.BlockSpec((E, D), full2d),
            pl.BlockSpec((E, 1), full2d),
            pl.BlockSpec((BN, 1), row_blk),
            pl.BlockSpec((D, D), full2d),
            pl.BlockSpec((1, D), full2d),
        ],
        out_specs=[
            pl.BlockSpec((BN, D), row_blk),
            pl.BlockSpec((E, D), full2d),
        ],
        out_shape=[
            jax.ShapeDtypeStruct((N, D), jnp.float32),
            jax.ShapeDtypeStruct((E, D), jnp.float32),
        ],
        scratch_shapes=[pltpu.VMEM((E, D), jnp.bfloat16)],
    )(hb, p1, de2d, s2d, W1, b1r)

    x_final = pl.pallas_call(
        _sweep3_body,
        grid=(NB,),
        in_specs=[
            pl.BlockSpec((BN, E), row_blk),
            pl.BlockSpec((E, D), full2d),
            pl.BlockSpec((E, 1), full2d),
            pl.BlockSpec((BN, 1), row_blk),
            pl.BlockSpec((D, D), full2d),
            pl.BlockSpec((1, D), full2d),
            pl.BlockSpec((BN, D), row_blk),
            pl.BlockSpec((BN, D), row_blk),
            pl.BlockSpec(memory_space=pltpu.SMEM),
        ],
        out_specs=pl.BlockSpec((BN, D), row_blk),
        out_shape=jax.ShapeDtypeStruct((N, D), jnp.float32),
        scratch_shapes=[pltpu.VMEM((E, D), jnp.bfloat16)],
    )(hb, p2, de2d, s2d, W2, b2r, x0, x1, w)

    return x_final


# speedup vs baseline: 1.0989x; 1.0989x over previous
import jax
import jax.numpy as jnp
from jax.experimental import pallas as pl
from jax.experimental.pallas import tpu as pltpu

N = 10000
E = 4096
D = 128
T = 5
BN = 400
NB = N // BN
BLOCKS_PER_TYPE = (N // T) // BN

_CONTRACT_ROWS = (((0,), (0,)), ((), ()))
_ROW_BLK = lambda i: (i, 0)
_FULL2D = lambda i: (0, 0)


def _sweep1_body(emb_ref, h_ref, s_ref, pw_ref, pb_ref,
                 x0_ref, hbs_ref, p1t_ref):
    i = pl.program_id(0)
    x0 = jax.lax.dot_general(
        emb_ref[...], pw_ref[0],
        (((1,), (0,)), ((), ())),
        preferred_element_type=jnp.float32,
        precision=jax.lax.Precision.HIGHEST) + pb_ref[0]
    x0_ref[...] = x0
    hbs = (h_ref[...] * s_ref[...]).astype(jnp.bfloat16)
    hbs_ref[...] = hbs

    @pl.when(i == 0)
    def _():
        p1t_ref[...] = jnp.zeros_like(p1t_ref)

    p1t_ref[...] += jax.lax.dot_general(
        x0.astype(jnp.bfloat16), hbs, _CONTRACT_ROWS,
        preferred_element_type=jnp.float32)


def _scale_transpose_body(pt_ref, de_ref, m_ref):
    m_ref[...] = jnp.transpose((pt_ref[...] * de_ref[...]).astype(jnp.bfloat16))


def _sweep2_body(hbs_ref, m1_ref, w1_ref, b1_ref, x1_ref, p2t_ref):
    i = pl.program_id(0)
    hbs = hbs_ref[...]
    y = jax.lax.dot_general(
        hbs, m1_ref[...], (((1,), (0,)), ((), ())),
        preferred_element_type=jnp.float32)
    x1 = jax.lax.dot_general(
        y, w1_ref[...], (((1,), (0,)), ((), ())),
        preferred_element_type=jnp.float32,
        precision=jax.lax.Precision.HIGHEST) + b1_ref[...]
    x1_ref[...] = x1

    @pl.when(i == 0)
    def _():
        p2t_ref[...] = jnp.zeros_like(p2t_ref)

    p2t_ref[...] += jax.lax.dot_general(
        x1.astype(jnp.bfloat16), hbs, _CONTRACT_ROWS,
        preferred_element_type=jnp.float32)


def _sweep3_body(hbs_ref, m2_ref, w2_ref, b2_ref, x0_ref, x1_ref, wf_ref,
                 out_ref):
    y = jax.lax.dot_general(
        hbs_ref[...], m2_ref[...], (((1,), (0,)), ((), ())),
        preferred_element_type=jnp.float32)
    x2 = jax.lax.dot_general(
        y, w2_ref[...], (((1,), (0,)), ((), ())),
        preferred_element_type=jnp.float32,
        precision=jax.lax.Precision.HIGHEST) + b2_ref[...]
    out_ref[...] = (wf_ref[0] * x0_ref[...] + wf_ref[1] * x1_ref[...]
                    + wf_ref[2] * x2)


def _scale_transpose(pt, de_row):
    return pl.pallas_call(
        _scale_transpose_body,
        grid=(1,),
        in_specs=[pl.BlockSpec((D, E), _FULL2D),
                  pl.BlockSpec((1, E), _FULL2D)],
        out_specs=pl.BlockSpec((E, D), _FULL2D),
        out_shape=jax.ShapeDtypeStruct((E, D), jnp.bfloat16),
    )(pt, de_row)


def kernel(H, Dv_inv_sqrt, De_inv, emb, projW, projB, W1, b1, W2, b2, fusion):
    s2d = Dv_inv_sqrt.reshape(N, 1)
    de_row = De_inv.reshape(1, E)
    pb3d = projB.reshape(T, 1, D)
    b1r = b1.reshape(1, D)
    b2r = b2.reshape(1, D)
    w = jax.nn.softmax(fusion, axis=0)

    x0, hbs, p1t = pl.pallas_call(
        _sweep1_body,
        grid=(NB,),
        in_specs=[
            pl.BlockSpec((BN, D), _ROW_BLK),
            pl.BlockSpec((BN, E), _ROW_BLK),
            pl.BlockSpec((BN, 1), _ROW_BLK),
            pl.BlockSpec((1, D, D), lambda i: (i // BLOCKS_PER_TYPE, 0, 0)),
            pl.BlockSpec((1, 1, D), lambda i: (i // BLOCKS_PER_TYPE, 0, 0)),
        ],
        out_specs=[
            pl.BlockSpec((BN, D), _ROW_BLK),
            pl.BlockSpec((BN, E), _ROW_BLK),
            pl.BlockSpec((D, E), _FULL2D),
        ],
        out_shape=[
            jax.ShapeDtypeStruct((N, D), jnp.float32),
            jax.ShapeDtypeStruct((N, E), jnp.bfloat16),
            jax.ShapeDtypeStruct((D, E), jnp.float32),
        ],
    )(emb, H, s2d, projW, pb3d)

    m1 = _scale_transpose(p1t, de_row)

    x1, p2t = pl.pallas_call(
        _sweep2_body,
        grid=(NB,),
        in_specs=[
            pl.BlockSpec((BN, E), _ROW_BLK),
            pl.BlockSpec((E, D), _FULL2D),
            pl.BlockSpec((D, D), _FULL2D),
            pl.BlockSpec((1, D), _FULL2D),
        ],
        out_specs=[
            pl.BlockSpec((BN, D), _ROW_BLK),
            pl.BlockSpec((D, E), _FULL2D),
        ],
        out_shape=[
            jax.ShapeDtypeStruct((N, D), jnp.float32),
            jax.ShapeDtypeStruct((D, E), jnp.float32),
        ],
    )(hbs, m1, W1, b1r)

    m2 = _scale_transpose(p2t, de_row)

    x_final = pl.pallas_call(
        _sweep3_body,
        grid=(NB,),
        in_specs=[
            pl.BlockSpec((BN, E), _ROW_BLK),
            pl.BlockSpec((E, D), _FULL2D),
            pl.BlockSpec((D, D), _FULL2D),
            pl.BlockSpec((1, D), _FULL2D),
            pl.BlockSpec((BN, D), _ROW_BLK),
            pl.BlockSpec((BN, D), _ROW_BLK),
            pl.BlockSpec(memory_space=pltpu.SMEM),
        ],
        out_specs=pl.BlockSpec((BN, D), _ROW_BLK),
        out_shape=jax.ShapeDtypeStruct((N, D), jnp.float32),
    )(hbs, m2, W2, b2r, x0, x1, w)

    return x_final


# speedup vs baseline: 1.1197x; 1.0189x over previous
import jax
import jax.numpy as jnp
from jax.experimental import pallas as pl
from jax.experimental.pallas import tpu as pltpu

N = 10000
E = 4096
D = 128
T = 5
BN = 400
NB = N // BN
BLOCKS_PER_TYPE = (N // T) // BN

_CONTRACT_ROWS = (((0,), (0,)), ((), ()))
_ROW_BLK = lambda i: (i, 0)
_FULL2D = lambda i: (0, 0)


def _sweep1_body(emb_ref, h_ref, s_ref, pw_ref, pb_ref,
                 x0_ref, hi_ref, p1t_ref):
    i = pl.program_id(0)
    x0 = jax.lax.dot_general(
        emb_ref[...], pw_ref[0],
        (((1,), (0,)), ((), ())),
        preferred_element_type=jnp.float32,
        precision=jax.lax.Precision.HIGHEST) + pb_ref[0]
    x0_ref[...] = x0
    h = h_ref[...]
    hi_ref[...] = h.astype(jnp.int8)

    @pl.when(i == 0)
    def _():
        p1t_ref[...] = jnp.zeros_like(p1t_ref)

    p1t_ref[...] += jax.lax.dot_general(
        (x0 * s_ref[...]).astype(jnp.bfloat16), h.astype(jnp.bfloat16),
        _CONTRACT_ROWS, preferred_element_type=jnp.float32)


def _scale_transpose_body(pt_ref, de_ref, m_ref):
    m_ref[...] = jnp.transpose((pt_ref[...] * de_ref[...]).astype(jnp.bfloat16))


def _sweep2_body(hi_ref, s_ref, m1_ref, w1_ref, b1_ref, x1_ref, p2t_ref):
    i = pl.program_id(0)
    hb = hi_ref[...].astype(jnp.bfloat16)
    s = s_ref[...]
    y = jax.lax.dot_general(
        hb, m1_ref[...], (((1,), (0,)), ((), ())),
        preferred_element_type=jnp.float32)
    x1 = jax.lax.dot_general(
        y * s, w1_ref[...], (((1,), (0,)), ((), ())),
        preferred_element_type=jnp.float32,
        precision=jax.lax.Precision.HIGHEST) + b1_ref[...]
    x1_ref[...] = x1

    @pl.when(i == 0)
    def _():
        p2t_ref[...] = jnp.zeros_like(p2t_ref)

    p2t_ref[...] += jax.lax.dot_general(
        (x1 * s).astype(jnp.bfloat16), hb, _CONTRACT_ROWS,
        preferred_element_type=jnp.float32)


def _sweep3_body(hi_ref, s_ref, m2_ref, w2_ref, b2_ref, x0_ref, x1_ref,
                 wf_ref, out_ref):
    y = jax.lax.dot_general(
        hi_ref[...].astype(jnp.bfloat16), m2_ref[...],
        (((1,), (0,)), ((), ())),
        preferred_element_type=jnp.float32)
    x2 = jax.lax.dot_general(
        y * s_ref[...], w2_ref[...], (((1,), (0,)), ((), ())),
        preferred_element_type=jnp.float32,
        precision=jax.lax.Precision.HIGHEST) + b2_ref[...]
    out_ref[...] = (wf_ref[0] * x0_ref[...] + wf_ref[1] * x1_ref[...]
                    + wf_ref[2] * x2)


def _scale_transpose(pt, de_row):
    return pl.pallas_call(
        _scale_transpose_body,
        grid=(1,),
        in_specs=[pl.BlockSpec((D, E), _FULL2D),
                  pl.BlockSpec((1, E), _FULL2D)],
        out_specs=pl.BlockSpec((E, D), _FULL2D),
        out_shape=jax.ShapeDtypeStruct((E, D), jnp.bfloat16),
    )(pt, de_row)


def kernel(H, Dv_inv_sqrt, De_inv, emb, projW, projB, W1, b1, W2, b2, fusion):
    s2d = Dv_inv_sqrt.reshape(N, 1)
    de_row = De_inv.reshape(1, E)
    pb3d = projB.reshape(T, 1, D)
    b1r = b1.reshape(1, D)
    b2r = b2.reshape(1, D)
    w = jax.nn.softmax(fusion, axis=0)

    x0, hi, p1t = pl.pallas_call(
        _sweep1_body,
        grid=(NB,),
        in_specs=[
            pl.BlockSpec((BN, D), _ROW_BLK),
            pl.BlockSpec((BN, E), _ROW_BLK),
            pl.BlockSpec((BN, 1), _ROW_BLK),
            pl.BlockSpec((1, D, D), lambda i: (i // BLOCKS_PER_TYPE, 0, 0)),
            pl.BlockSpec((1, 1, D), lambda i: (i // BLOCKS_PER_TYPE, 0, 0)),
        ],
        out_specs=[
            pl.BlockSpec((BN, D), _ROW_BLK),
            pl.BlockSpec((BN, E), _ROW_BLK),
            pl.BlockSpec((D, E), _FULL2D),
        ],
        out_shape=[
            jax.ShapeDtypeStruct((N, D), jnp.float32),
            jax.ShapeDtypeStruct((N, E), jnp.int8),
            jax.ShapeDtypeStruct((D, E), jnp.float32),
        ],
    )(emb, H, s2d, projW, pb3d)

    m1 = _scale_transpose(p1t, de_row)

    x1, p2t = pl.pallas_call(
        _sweep2_body,
        grid=(NB,),
        in_specs=[
            pl.BlockSpec((BN, E), _ROW_BLK),
            pl.BlockSpec((BN, 1), _ROW_BLK),
            pl.BlockSpec((E, D), _FULL2D),
            pl.BlockSpec((D, D), _FULL2D),
            pl.BlockSpec((1, D), _FULL2D),
        ],
        out_specs=[
            pl.BlockSpec((BN, D), _ROW_BLK),
            pl.BlockSpec((D, E), _FULL2D),
        ],
        out_shape=[
            jax.ShapeDtypeStruct((N, D), jnp.float32),
            jax.ShapeDtypeStruct((D, E), jnp.float32),
        ],
    )(hi, s2d, m1, W1, b1r)

    m2 = _scale_transpose(p2t, de_row)

    x_final = pl.pallas_call(
        _sweep3_body,
        grid=(NB,),
        in_specs=[
            pl.BlockSpec((BN, E), _ROW_BLK),
            pl.BlockSpec((BN, 1), _ROW_BLK),
            pl.BlockSpec((E, D), _FULL2D),
            pl.BlockSpec((D, D), _FULL2D),
            pl.BlockSpec((1, D), _FULL2D),
            pl.BlockSpec((BN, D), _ROW_BLK),
            pl.BlockSpec((BN, D), _ROW_BLK),
            pl.BlockSpec(memory_space=pltpu.SMEM),
        ],
        out_specs=pl.BlockSpec((BN, D), _ROW_BLK),
        out_shape=jax.ShapeDtypeStruct((N, D), jnp.float32),
    )(hi, s2d, m2, W2, b2r, x0, x1, w)

    return x_final
